# baseline (speedup 1.0000x reference)
import functools

import jax
import jax.numpy as jnp
from jax import lax
from jax.experimental import pallas as pl
from jax.experimental.pallas import tpu as pltpu

_WINDOW = 512
_N_FRAMES = 18750
_BLK = 592
_N_BLOCKS = 32
_N_PAD = _BLK * _N_BLOCKS


def _score_body(x_ref, w_ref, b_ref, pdest_ref, carry_ref):
    k = pl.program_id(0)
    frames = x_ref[...]
    w = w_ref[...]
    logits = jnp.dot(frames, w, preferred_element_type=jnp.float32)
    logits = logits + b_ref[0]
    row = lax.broadcasted_iota(jnp.int32, (_BLK, 1), 0) + k * _BLK
    flag = jnp.where((logits > 0.0) & (row < _N_FRAMES), 1.0, 0.0)
    i = lax.broadcasted_iota(jnp.int32, (_BLK, _BLK), 0)
    j = lax.broadcasted_iota(jnp.int32, (_BLK, _BLK), 1)
    tri = jnp.where(i >= j, 1.0, 0.0)
    csum = jnp.dot(tri, flag, preferred_element_type=jnp.float32)

    @pl.when(k == 0)
    def _():
        carry_ref[0, 0] = 0.0

    carry = carry_ref[0, 0]
    dest = jnp.where(flag > 0.0, carry + csum, 0.0)
    pdest_ref[...] = dest.astype(jnp.int32).reshape(1, 1, _BLK)
    carry_ref[0, 0] = carry + jnp.max(csum)


def _score(x2d, w, b):
    pdest = pl.pallas_call(
        _score_body,
        grid=(_N_BLOCKS,),
        in_specs=[
            pl.BlockSpec((_BLK, _WINDOW), lambda k: (k, 0)),
            pl.BlockSpec((_WINDOW, 1), lambda k: (0, 0)),
            pl.BlockSpec(memory_space=pltpu.SMEM),
        ],
        out_specs=pl.BlockSpec((1, 1, _BLK), lambda k: (k, 0, 0)),
        out_shape=jax.ShapeDtypeStruct((_N_BLOCKS, 1, _BLK), jnp.int32),
        scratch_shapes=[pltpu.SMEM((1, 1), jnp.float32)],
    )(x2d, w.reshape(_WINDOW, 1), b.reshape(1))
    return pdest.reshape(_N_PAD)


def kernel(x, W, b):
    x2d = x[: _N_FRAMES * _WINDOW].reshape(_N_FRAMES, _WINDOW)
    pdest = _score(x2d, W, b)
    flags = pdest[:_N_FRAMES] > 0
    nsp = jnp.sum(flags)
    idxs = jnp.nonzero(flags, size=_N_FRAMES, fill_value=0)[0]
    valid = (jnp.arange(_N_FRAMES) < nsp).astype(x2d.dtype)
    out = jnp.take(x2d, idxs, axis=0) * valid[:, None]
    return out.reshape(-1)
